# Initial kernel scaffold; baseline (speedup 1.0000x reference)
import jax
import jax.numpy as jnp
from jax.experimental import pallas as pl


def kernel(pos_i, pos_j, distance, logits):
    raise NotImplementedError("write your pallas kernel here")


# speedup vs baseline: 137.1165x; 137.1165x over previous
import functools

import jax
import jax.numpy as jnp
from jax import lax
from jax.experimental import pallas as pl
from jax.experimental.pallas import tpu as pltpu
from jax.experimental.pallas import tpu_sc as plsc

B, P, F = 16384, 676, 26
N = B * P
NC, NS, L = 2, 16, 16
NW = NC * NS
PER_W = N // NW
CH = 2048
NCH = PER_W // CH
NSTEP = CH // L
TAB = 688
TSTEPS = TAB // L


def _sc_body(pi_hbm, pj_hbm, d_hbm, lg_hbm, out_hbm, tab, pib, pjb, db, ob):
    pltpu.sync_copy(lg_hbm, tab)

    def sig(t, c):
        sl = pl.ds(t * L, L)
        tab[sl] = 1.0 / (1.0 + jnp.exp(-tab[sl]))
        return c

    lax.fori_loop(0, TSTEPS, sig, 0)

    wid = lax.axis_index("s") * NC + lax.axis_index("c")
    base = wid * PER_W

    def chunk(c, carry):
        off = base + c * CH
        pltpu.sync_copy(pi_hbm.at[pl.ds(off, CH)], pib)
        pltpu.sync_copy(pj_hbm.at[pl.ds(off, CH)], pjb)
        pltpu.sync_copy(d_hbm.at[pl.ds(off, CH)], db)

        def step(s, cc):
            sl = pl.ds(s * L, L)
            idx = pib[sl] * F + pjb[sl]
            g = plsc.load_gather(tab, [idx])
            ob[sl] = g * jnp.exp(-db[sl])
            return cc

        lax.fori_loop(0, NSTEP, step, 0)
        pltpu.sync_copy(ob, out_hbm.at[pl.ds(off, CH)])
        return carry

    lax.fori_loop(0, NCH, chunk, 0)


@jax.jit
def kernel(pos_i, pos_j, distance, logits):
    pi = pos_i.reshape(N).astype(jnp.int32)
    pj = pos_j.reshape(N).astype(jnp.int32)
    d = distance.reshape(N)
    lg = jnp.pad(logits.reshape(F * F), (0, TAB - F * F))

    mesh = plsc.VectorSubcoreMesh(
        core_axis_name="c", subcore_axis_name="s", num_cores=NC, num_subcores=NS
    )
    out = pl.kernel(
        _sc_body,
        out_type=jax.ShapeDtypeStruct((N,), jnp.float32),
        mesh=mesh,
        scratch_types=[
            pltpu.VMEM((TAB,), jnp.float32),
            pltpu.VMEM((CH,), jnp.int32),
            pltpu.VMEM((CH,), jnp.int32),
            pltpu.VMEM((CH,), jnp.float32),
            pltpu.VMEM((CH,), jnp.float32),
        ],
        compiler_params=pltpu.CompilerParams(needs_layout_passes=False),
    )(pi, pj, d, lg)
    return out.reshape(B, P)


# speedup vs baseline: 270.8838x; 1.9756x over previous
import functools

import jax
import jax.numpy as jnp
from jax import lax
from jax.experimental import pallas as pl
from jax.experimental.pallas import tpu as pltpu
from jax.experimental.pallas import tpu_sc as plsc

B, P, F = 16384, 676, 26
N = B * P
NC, NS, L = 2, 16, 16
NW = NC * NS
PER_W = N // NW
CH = 6656
NCH = PER_W // CH
TAB = 688


def _sc_body(pi_hbm, pj_hbm, d_hbm, lg_hbm, out_hbm, tab,
             pib0, pjb0, db0, ob0, pib1, pjb1, db1, ob1,
             si0, si1, so0, so1):
    pltpu.sync_copy(lg_hbm, tab)

    @plsc.parallel_loop(0, TAB, step=L)
    def _(t):
        sl = pl.ds(t, L)
        tab[sl] = 1.0 / (1.0 + jnp.exp(-tab[sl]))

    wid = lax.axis_index("s") * NC + lax.axis_index("c")
    base = wid * PER_W

    ins = ((pib0, pjb0, db0, si0), (pib1, pjb1, db1, si1))
    outs = ((ob0, so0), (ob1, so1))

    def start_in(b, c):
        off = base + c * CH
        pib, pjb, db, sem = ins[b]
        pltpu.async_copy(pi_hbm.at[pl.ds(off, CH)], pib, sem)
        pltpu.async_copy(pj_hbm.at[pl.ds(off, CH)], pjb, sem)
        pltpu.async_copy(d_hbm.at[pl.ds(off, CH)], db, sem)

    def wait_in(b):
        pib, pjb, db, sem = ins[b]
        pltpu.make_async_copy(pi_hbm.at[pl.ds(0, CH)], pib, sem).wait()
        pltpu.make_async_copy(pj_hbm.at[pl.ds(0, CH)], pjb, sem).wait()
        pltpu.make_async_copy(d_hbm.at[pl.ds(0, CH)], db, sem).wait()

    def start_out(b, c):
        off = base + c * CH
        ob, sem = outs[b]
        pltpu.async_copy(ob, out_hbm.at[pl.ds(off, CH)], sem)

    def wait_out(b):
        ob, sem = outs[b]
        pltpu.make_async_copy(ob, out_hbm.at[pl.ds(0, CH)], sem).wait()

    def compute(b):
        pib, pjb, db, _ = ins[b]
        ob, _ = outs[b]

        @plsc.parallel_loop(0, CH, step=L, unroll=8)
        def _(i):
            sl = pl.ds(i, L)
            idx = pib[sl] * F + pjb[sl]
            g = plsc.load_gather(tab, [idx])
            ob[sl] = g * jnp.exp(-db[sl])

    start_in(0, 0)

    def group(gg, carry):
        for b in range(2):
            c = gg * 2 + b

            @pl.when(c + 1 < NCH)
            def _():
                start_in(1 - b, c + 1)

            wait_in(b)

            @pl.when(gg > 0)
            def _():
                wait_out(b)

            compute(b)
            start_out(b, c)
        return carry

    lax.fori_loop(0, NCH // 2, group, 0)
    wait_out(0)
    wait_out(1)


@jax.jit
def kernel(pos_i, pos_j, distance, logits):
    pi = pos_i.reshape(N).astype(jnp.int32)
    pj = pos_j.reshape(N).astype(jnp.int32)
    d = distance.reshape(N)
    lg = jnp.pad(logits.reshape(F * F), (0, TAB - F * F))

    mesh = plsc.VectorSubcoreMesh(
        core_axis_name="c", subcore_axis_name="s", num_cores=NC, num_subcores=NS
    )
    out = pl.kernel(
        _sc_body,
        out_type=jax.ShapeDtypeStruct((N,), jnp.float32),
        mesh=mesh,
        scratch_types=[
            pltpu.VMEM((TAB,), jnp.float32),
            pltpu.VMEM((CH,), jnp.int32),
            pltpu.VMEM((CH,), jnp.int32),
            pltpu.VMEM((CH,), jnp.float32),
            pltpu.VMEM((CH,), jnp.float32),
            pltpu.VMEM((CH,), jnp.int32),
            pltpu.VMEM((CH,), jnp.int32),
            pltpu.VMEM((CH,), jnp.float32),
            pltpu.VMEM((CH,), jnp.float32),
            pltpu.SemaphoreType.DMA,
            pltpu.SemaphoreType.DMA,
            pltpu.SemaphoreType.DMA,
            pltpu.SemaphoreType.DMA,
        ],
        compiler_params=pltpu.CompilerParams(needs_layout_passes=False),
    )(pi, pj, d, lg)
    return out.reshape(B, P)


# speedup vs baseline: 432.0426x; 1.5949x over previous
import jax
import jax.numpy as jnp
from jax import lax
from jax.experimental import pallas as pl
from jax.experimental.pallas import tpu as pltpu
from jax.experimental.pallas import tpu_sc as plsc

B, P, F = 16384, 676, 26
NC, NS, L = 2, 16, 16
NW = NC * NS
ROWS_W = B // NW
RWS = 16
NCH = ROWS_W // RWS
NFULL = (P // L) * L - L
TAIL = P - L
TAB = 688


def _sc_body(pi_hbm, pj_hbm, d_hbm, lg_hbm, out_hbm, tab,
             pib0, pjb0, db0, ob0, pib1, pjb1, db1, ob1,
             si0, si1, so0, so1):
    pltpu.sync_copy(lg_hbm, tab)

    @plsc.parallel_loop(0, TAB, step=L)
    def _(t):
        sl = pl.ds(t, L)
        tab[sl] = 1.0 / (1.0 + jnp.exp(-tab[sl]))

    wid = lax.axis_index("s") * NC + lax.axis_index("c")
    base = wid * ROWS_W

    ins = ((pib0, pjb0, db0, si0), (pib1, pjb1, db1, si1))
    outs = ((ob0, so0), (ob1, so1))

    def start_in(b, c):
        r0 = base + c * RWS
        pib, pjb, db, sem = ins[b]
        pltpu.async_copy(pi_hbm.at[pl.ds(r0, RWS)], pib, sem)
        pltpu.async_copy(pj_hbm.at[pl.ds(r0, RWS)], pjb, sem)
        pltpu.async_copy(d_hbm.at[pl.ds(r0, RWS)], db, sem)

    def wait_in(b):
        pib, pjb, db, sem = ins[b]
        pltpu.make_async_copy(pi_hbm.at[pl.ds(0, RWS)], pib, sem).wait()
        pltpu.make_async_copy(pj_hbm.at[pl.ds(0, RWS)], pjb, sem).wait()
        pltpu.make_async_copy(d_hbm.at[pl.ds(0, RWS)], db, sem).wait()

    def start_out(b, c):
        r0 = base + c * RWS
        ob, sem = outs[b]
        pltpu.async_copy(ob, out_hbm.at[pl.ds(r0, RWS)], sem)

    def wait_out(b):
        ob, sem = outs[b]
        pltpu.make_async_copy(ob, out_hbm.at[pl.ds(0, RWS)], sem).wait()

    def compute(b):
        pib, pjb, db, _ = ins[b]
        ob, _ = outs[b]

        def row(r, carry):
            def cell(i):
                sl = pl.ds(i, L)
                idx = pib[r, sl] * F + pjb[r, sl]
                g = plsc.load_gather(tab, [idx])
                ob[r, sl] = g * jnp.exp(-db[r, sl])

            loop = plsc.parallel_loop(0, NFULL + L, step=L, unroll=7)
            loop(cell)
            cell(TAIL)
            return carry

        lax.fori_loop(0, RWS, row, 0)

    start_in(0, 0)

    def group(gg, carry):
        for b in range(2):
            c = gg * 2 + b

            @pl.when(c + 1 < NCH)
            def _():
                start_in(1 - b, c + 1)

            wait_in(b)

            @pl.when(gg > 0)
            def _():
                wait_out(b)

            compute(b)
            start_out(b, c)
        return carry

    lax.fori_loop(0, NCH // 2, group, 0)
    wait_out(0)
    wait_out(1)


@jax.jit
def kernel(pos_i, pos_j, distance, logits):
    pi = pos_i.astype(jnp.int32)
    pj = pos_j.astype(jnp.int32)
    lg = jnp.pad(logits.reshape(F * F), (0, TAB - F * F))

    mesh = plsc.VectorSubcoreMesh(
        core_axis_name="c", subcore_axis_name="s", num_cores=NC, num_subcores=NS
    )
    return pl.kernel(
        _sc_body,
        out_type=jax.ShapeDtypeStruct((B, P), jnp.float32),
        mesh=mesh,
        scratch_types=[
            pltpu.VMEM((TAB,), jnp.float32),
            pltpu.VMEM((RWS, P), jnp.int32),
            pltpu.VMEM((RWS, P), jnp.int32),
            pltpu.VMEM((RWS, P), jnp.float32),
            pltpu.VMEM((RWS, P), jnp.float32),
            pltpu.VMEM((RWS, P), jnp.int32),
            pltpu.VMEM((RWS, P), jnp.int32),
            pltpu.VMEM((RWS, P), jnp.float32),
            pltpu.VMEM((RWS, P), jnp.float32),
            pltpu.SemaphoreType.DMA,
            pltpu.SemaphoreType.DMA,
            pltpu.SemaphoreType.DMA,
            pltpu.SemaphoreType.DMA,
        ],
        compiler_params=pltpu.CompilerParams(needs_layout_passes=False),
    )(pi, pj, distance, lg)
